# baseline (speedup 1.0000x reference)
import functools

import jax
import jax.numpy as jnp
from jax import lax
from jax.experimental import pallas as pl
from jax.experimental.pallas import tpu as pltpu
from jax.experimental.pallas import tpu_sc as plsc

ROWS, COLS = 2048, 4096
NUM_ZERO = 2048
LANES = 16
NCHUNK = COLS // LANES
NWORKERS = 32
ROWS_PER_WORKER = ROWS // NWORKERS
NPIV = 15
GROUPS = 4
GCH = NCHUNK // GROUPS
BLK = GCH * LANES + LANES
UNROLL = 8
SIG_SCALE = 0.8 / (2.0 / 3.0)
TC_BLOCK_ROWS = 128
I32_MIN = -2147483648
I32_MAX = 2147483647
MAX_ROUNDS = 64


def _keys_from_bits(b):
    return b ^ ((b >> 31) & 0x7FFFFFFF)


def _where_chain(t, counts, lo_vals, hi_vals, below_vals):
    lo = lo_vals[-1]
    hi = hi_vals[-1]
    below = below_vals[-1]
    for c, lv, hv, bv in zip(reversed(counts), reversed(lo_vals[:-1]),
                             reversed(hi_vals[:-1]), reversed(below_vals[:-1])):
        cond = t <= c
        lo = jnp.where(cond, lv, lo)
        hi = jnp.where(cond, hv, hi)
        below = jnp.where(cond, bv, below)
    return lo, hi, below


def _sc_body(z_hbm, thr_hbm, row_v, key_v, cand_a, cand_b, out_v):
    cid = lax.axis_index("c")
    sid = lax.axis_index("s")
    wid = sid * 2 + cid
    base_row = wid * ROWS_PER_WORKER
    zeros = jnp.zeros((LANES,), jnp.int32)
    lane_iota = lax.iota(jnp.int32, LANES)

    def select_round(src, dst, n, t):
        sample = src[pl.ds(0, LANES)]
        ss, _ = plsc.sort_key_val(sample, sample)
        p1, p2, p3 = ss[4], ss[8], ss[12]
        nch = (n + LANES - 1) >> 4

        def cbody(j, accs):
            a1, a2, a3, e1, e2, e3 = accs
            k = src[pl.ds(j * LANES, LANES)]
            k = jnp.where(lane_iota < (n - j * LANES), k, I32_MAX)
            a1 = a1 + (k < p1).astype(jnp.int32)
            a2 = a2 + (k < p2).astype(jnp.int32)
            a3 = a3 + (k < p3).astype(jnp.int32)
            e1 = e1 + (k == p1).astype(jnp.int32)
            e2 = e2 + (k == p2).astype(jnp.int32)
            e3 = e3 + (k == p3).astype(jnp.int32)
            return a1, a2, a3, e1, e2, e3

        accs = lax.fori_loop(0, nch, cbody, (zeros,) * 6)
        clt1, clt2, clt3 = (jnp.sum(a) for a in accs[:3])
        ceq1, ceq2, ceq3 = (jnp.sum(a) for a in accs[3:])
        cle1 = clt1 + ceq1
        cle2 = clt2 + ceq2
        cle3 = clt3 + ceq3

        in1 = jnp.logical_and(t > clt1, t <= cle1)
        in2 = jnp.logical_and(t > clt2, t <= cle2)
        in3 = jnp.logical_and(t > clt3, t <= cle3)
        done = jnp.logical_or(in1, jnp.logical_or(in2, in3))
        ans = jnp.where(t <= cle1, p1, jnp.where(t <= cle2, p2, p3))
        lo = jnp.where(t <= clt1, I32_MIN,
                       jnp.where(t <= clt2, p1,
                                 jnp.where(t <= clt3, p2, p3)))
        hi = jnp.where(t <= clt1, p1,
                       jnp.where(t <= clt2, p2,
                                 jnp.where(t <= clt3, p3, I32_MAX)))
        below = jnp.where(t <= clt1, 0,
                          jnp.where(t <= clt2, cle1,
                                    jnp.where(t <= clt3, cle2, cle3)))

        def compact(j, ptr):
            k = src[pl.ds(j * LANES, LANES)]
            m = jnp.logical_and(lane_iota < (n - j * LANES),
                                jnp.logical_and(k > lo, k < hi))
            plsc.store_compressed(dst.at[pl.ds(ptr, LANES)], k, mask=m)
            return ptr + plsc.all_reduce_population_count(m)[0]

        ptr = lax.fori_loop(0, nch, compact, jnp.int32(0))
        n2 = jnp.where(done, 1, ptr)
        t2 = jnp.where(done, 1, t - below)
        return n2, t2, done.astype(jnp.int32), ans

    def row_threshold(row):
        pltpu.sync_copy(z_hbm.at[row], row_v)
        t0 = jnp.int32(NUM_ZERO)

        z0 = row_v[pl.ds(0, LANES)]
        k0 = _keys_from_bits(lax.bitcast_convert_type(z0, jnp.int32))
        ss, _ = plsc.sort_key_val(k0, k0)
        pivs = [ss[i] for i in range(1, NPIV + 1)]

        def pass1(jo, accs):
            accs = list(accs)
            for ji in range(UNROLL):
                j = jo * UNROLL + ji
                z = row_v[pl.ds(j * LANES, LANES)]
                b = lax.bitcast_convert_type(z, jnp.int32)
                k = _keys_from_bits(b)
                key_v[pl.ds(j * LANES, LANES)] = k
                for i in range(NPIV):
                    accs[i] = accs[i] + (k < pivs[i]).astype(jnp.int32)
            return tuple(accs)

        accs = lax.fori_loop(0, NCHUNK // UNROLL, pass1, (zeros,) * NPIV)
        clts = [jnp.sum(a) for a in accs]

        lo, hi, below = _where_chain(
            t0, clts,
            [I32_MIN] + pivs,
            pivs + [I32_MAX],
            [jnp.int32(0)] + clts,
        )
        t = t0 - below

        def compact1(jj, ptrs):
            new_ptrs = []
            for gi in range(GROUPS):
                ptr = ptrs[gi]
                j = gi * GCH + jj
                k = key_v[pl.ds(j * LANES, LANES)]
                m = jnp.logical_and(k >= lo, k < hi)
                plsc.store_compressed(
                    cand_b.at[pl.ds(gi * BLK + ptr, LANES)], k, mask=m)
                new_ptrs.append(
                    ptr + plsc.all_reduce_population_count(m)[0])
            return tuple(new_ptrs)

        ptrs = lax.fori_loop(
            0, GCH, compact1, (jnp.int32(0),) * GROUPS)

        q = jnp.int32(0)
        for gi in range(GROUPS):
            s_gi = ptrs[gi]

            def mcopy(c, _, gi=gi, q=q):
                cand_a[pl.ds(q + c * LANES, LANES)] = \
                    cand_b[pl.ds(gi * BLK + c * LANES, LANES)]
                return 0

            lax.fori_loop(0, (s_gi + LANES - 1) >> 4, mcopy, 0)
            q = q + s_gi
        n = q

        def wcond(c):
            n, t, done, ans, it = c
            return jnp.logical_and(
                done == 0, jnp.logical_and(n > LANES, it < MAX_ROUNDS))

        def wbody(c):
            n, t, done, ans, it = c
            n2, t2, d2, a2 = select_round(cand_a, cand_b, n, t)

            def copyb(j, _):
                cand_a[pl.ds(j * LANES, LANES)] = \
                    cand_b[pl.ds(j * LANES, LANES)]
                return 0

            lax.fori_loop(0, (n2 + LANES - 1) >> 4, copyb, 0)
            return n2, t2, d2, jnp.where(d2 == 1, a2, ans), it + 1

        n, t, done, ans, _ = lax.while_loop(
            wcond, wbody, (n, t, jnp.int32(0), jnp.int32(0), jnp.int32(0)))

        s = jnp.where(lane_iota < n, cand_a[pl.ds(0, LANES)], I32_MAX)
        ssort, _ = plsc.sort_key_val(s, s)
        pick = jnp.sum(jnp.where(lane_iota == (t - 1), ssort, 0))
        return jnp.where(done == 1, ans, pick)

    def group_body(g, _):
        def row_body(i, acc):
            kth = row_threshold(base_row + g * LANES + i)
            return jnp.where(lane_iota == i, kth, acc)

        acc = lax.fori_loop(0, LANES, row_body, zeros)
        out_v[pl.ds(g * LANES, LANES)] = acc
        return 0

    lax.fori_loop(0, ROWS_PER_WORKER // LANES, group_body, 0)
    pltpu.sync_copy(out_v, thr_hbm.at[pl.ds(base_row, ROWS_PER_WORKER)])


_sc_thresholds = functools.partial(
    pl.kernel,
    mesh=plsc.VectorSubcoreMesh(core_axis_name="c", subcore_axis_name="s"),
    out_type=jax.ShapeDtypeStruct((ROWS,), jnp.int32),
    compiler_params=pltpu.CompilerParams(needs_layout_passes=False),
    scratch_types=[
        pltpu.VMEM((COLS,), jnp.float32),
        pltpu.VMEM((COLS,), jnp.int32),
        pltpu.VMEM((COLS + LANES,), jnp.int32),
        pltpu.VMEM((GROUPS * BLK,), jnp.int32),
        pltpu.VMEM((ROWS_PER_WORKER,), jnp.int32),
    ],
)(_sc_body)


def _tc_body(z_ref, thr_ref, o_ref):
    z = z_ref[...]
    b = lax.bitcast_convert_type(z, jnp.int32)
    k = _keys_from_bits(b)
    kth = thr_ref[...]
    sig = jax.nn.sigmoid(z * jnp.float32(SIG_SCALE))
    o_ref[...] = jnp.where(k <= kth, jnp.float32(0.0), sig)


@jax.jit
def _impl(z):
    thr = _sc_thresholds(z)
    out = pl.pallas_call(
        _tc_body,
        grid=(ROWS // TC_BLOCK_ROWS,),
        in_specs=[
            pl.BlockSpec((TC_BLOCK_ROWS, COLS), lambda i: (i, 0)),
            pl.BlockSpec((TC_BLOCK_ROWS, 1), lambda i: (i, 0)),
        ],
        out_specs=pl.BlockSpec((TC_BLOCK_ROWS, COLS), lambda i: (i, 0)),
        out_shape=jax.ShapeDtypeStruct((ROWS, COLS), jnp.float32),
    )(z, thr.reshape(ROWS, 1))
    return out


def kernel(z_loga):
    return _impl(z_loga)


# speedup vs baseline: 2.9838x; 2.9838x over previous
import functools

import jax
import jax.numpy as jnp
from jax import lax
from jax.experimental import pallas as pl
from jax.experimental.pallas import tpu as pltpu
from jax.experimental.pallas import tpu_sc as plsc

ROWS, COLS = 2048, 4096
NUM_ZERO = 2048
LANES = 16
NCHUNK = COLS // LANES
NWORKERS = 32
ROWS_PER_WORKER = ROWS // NWORKERS
NPIV = 7
GROUPS = 4
GCH = NCHUNK // GROUPS
BLK = GCH * LANES + LANES
UNROLL = 4
SIG_SCALE = 0.8 / (2.0 / 3.0)
TC_BLOCK_ROWS = 128
I32_MIN = -2147483648
I32_MAX = 2147483647
MAX_ROUNDS = 64


def _keys_from_bits(b):
    return b ^ ((b >> 31) & 0x7FFFFFFF)


def _where_chain(t, counts, lo_vals, hi_vals, below_vals):
    lo = lo_vals[-1]
    hi = hi_vals[-1]
    below = below_vals[-1]
    for c, lv, hv, bv in zip(reversed(counts), reversed(lo_vals[:-1]),
                             reversed(hi_vals[:-1]), reversed(below_vals[:-1])):
        cond = t <= c
        lo = jnp.where(cond, lv, lo)
        hi = jnp.where(cond, hv, hi)
        below = jnp.where(cond, bv, below)
    return lo, hi, below


def _sc_body(z_hbm, thr_hbm, row_v, key_v, cand_a, cand_b, out_v):
    cid = lax.axis_index("c")
    sid = lax.axis_index("s")
    wid = sid * 2 + cid
    base_row = wid * ROWS_PER_WORKER
    zeros = jnp.zeros((LANES,), jnp.int32)
    lane_iota = lax.iota(jnp.int32, LANES)

    def select_round(src, dst, n, t):
        sample = src[pl.ds(0, LANES)]
        ss, _ = plsc.sort_key_val(sample, sample)
        p1, p2, p3 = ss[4], ss[8], ss[12]
        nch = (n + LANES - 1) >> 4

        def cbody(j, accs):
            a1, a2, a3, e1, e2, e3 = accs
            k = src[pl.ds(j * LANES, LANES)]
            k = jnp.where(lane_iota < (n - j * LANES), k, I32_MAX)
            a1 = a1 + (k < p1).astype(jnp.int32)
            a2 = a2 + (k < p2).astype(jnp.int32)
            a3 = a3 + (k < p3).astype(jnp.int32)
            e1 = e1 + (k == p1).astype(jnp.int32)
            e2 = e2 + (k == p2).astype(jnp.int32)
            e3 = e3 + (k == p3).astype(jnp.int32)
            return a1, a2, a3, e1, e2, e3

        accs = lax.fori_loop(0, nch, cbody, (zeros,) * 6)
        clt1, clt2, clt3 = (jnp.sum(a) for a in accs[:3])
        ceq1, ceq2, ceq3 = (jnp.sum(a) for a in accs[3:])
        cle1 = clt1 + ceq1
        cle2 = clt2 + ceq2
        cle3 = clt3 + ceq3

        in1 = jnp.logical_and(t > clt1, t <= cle1)
        in2 = jnp.logical_and(t > clt2, t <= cle2)
        in3 = jnp.logical_and(t > clt3, t <= cle3)
        done = jnp.logical_or(in1, jnp.logical_or(in2, in3))
        ans = jnp.where(t <= cle1, p1, jnp.where(t <= cle2, p2, p3))
        lo = jnp.where(t <= clt1, I32_MIN,
                       jnp.where(t <= clt2, p1,
                                 jnp.where(t <= clt3, p2, p3)))
        hi = jnp.where(t <= clt1, p1,
                       jnp.where(t <= clt2, p2,
                                 jnp.where(t <= clt3, p3, I32_MAX)))
        below = jnp.where(t <= clt1, 0,
                          jnp.where(t <= clt2, cle1,
                                    jnp.where(t <= clt3, cle2, cle3)))

        def compact(j, ptr):
            k = src[pl.ds(j * LANES, LANES)]
            m = jnp.logical_and(lane_iota < (n - j * LANES),
                                jnp.logical_and(k > lo, k < hi))
            plsc.store_compressed(dst.at[pl.ds(ptr, LANES)], k, mask=m)
            return ptr + plsc.all_reduce_population_count(m)[0]

        ptr = lax.fori_loop(0, nch, compact, jnp.int32(0))
        n2 = jnp.where(done, 1, ptr)
        t2 = jnp.where(done, 1, t - below)
        return n2, t2, done.astype(jnp.int32), ans

    def row_threshold(row):
        pltpu.sync_copy(z_hbm.at[row], row_v)
        t0 = jnp.int32(NUM_ZERO)

        z0 = row_v[pl.ds(0, LANES)]
        k0 = _keys_from_bits(lax.bitcast_convert_type(z0, jnp.int32))
        ss, _ = plsc.sort_key_val(k0, k0)
        pivs = [ss[2 * i] for i in range(1, NPIV + 1)]

        def pass1(jo, accs):
            accs = list(accs)
            for ji in range(UNROLL):
                j = jo * UNROLL + ji
                z = row_v[pl.ds(j * LANES, LANES)]
                b = lax.bitcast_convert_type(z, jnp.int32)
                k = _keys_from_bits(b)
                key_v[pl.ds(j * LANES, LANES)] = k
                for i in range(NPIV):
                    accs[i] = accs[i] + (k < pivs[i]).astype(jnp.int32)
            return tuple(accs)

        accs = lax.fori_loop(0, NCHUNK // UNROLL, pass1, (zeros,) * NPIV)
        clts = [jnp.sum(a) for a in accs]

        lo, hi, below = _where_chain(
            t0, clts,
            [I32_MIN] + pivs,
            pivs + [I32_MAX],
            [jnp.int32(0)] + clts,
        )
        t = t0 - below

        def compact1(jj, ptrs):
            new_ptrs = []
            for gi in range(GROUPS):
                ptr = ptrs[gi]
                j = gi * GCH + jj
                k = key_v[pl.ds(j * LANES, LANES)]
                m = jnp.logical_and(k >= lo, k < hi)
                plsc.store_compressed(
                    cand_b.at[pl.ds(gi * BLK + ptr, LANES)], k, mask=m)
                new_ptrs.append(
                    ptr + plsc.all_reduce_population_count(m)[0])
            return tuple(new_ptrs)

        ptrs = lax.fori_loop(
            0, GCH, compact1, (jnp.int32(0),) * GROUPS)

        q = jnp.int32(0)
        for gi in range(GROUPS):
            s_gi = ptrs[gi]

            def mcopy(c, _, gi=gi, q=q):
                cand_a[pl.ds(q + c * LANES, LANES)] = \
                    cand_b[pl.ds(gi * BLK + c * LANES, LANES)]
                return 0

            lax.fori_loop(0, (s_gi + LANES - 1) >> 4, mcopy, 0)
            q = q + s_gi
        n = q

        def wcond(c):
            n, t, done, ans, it = c
            return jnp.logical_and(
                done == 0, jnp.logical_and(n > LANES, it < MAX_ROUNDS))

        def wbody(c):
            n, t, done, ans, it = c
            n2, t2, d2, a2 = select_round(cand_a, cand_b, n, t)

            def copyb(j, _):
                cand_a[pl.ds(j * LANES, LANES)] = \
                    cand_b[pl.ds(j * LANES, LANES)]
                return 0

            lax.fori_loop(0, (n2 + LANES - 1) >> 4, copyb, 0)
            return n2, t2, d2, jnp.where(d2 == 1, a2, ans), it + 1

        n, t, done, ans, _ = lax.while_loop(
            wcond, wbody, (n, t, jnp.int32(0), jnp.int32(0), jnp.int32(0)))

        s = jnp.where(lane_iota < n, cand_a[pl.ds(0, LANES)], I32_MAX)
        ssort, _ = plsc.sort_key_val(s, s)
        pick = jnp.sum(jnp.where(lane_iota == (t - 1), ssort, 0))
        return jnp.where(done == 1, ans, pick)

    def group_body(g, _):
        def row_body(i, acc):
            kth = row_threshold(base_row + g * LANES + i)
            return jnp.where(lane_iota == i, kth, acc)

        acc = lax.fori_loop(0, LANES, row_body, zeros)
        out_v[pl.ds(g * LANES, LANES)] = acc
        return 0

    lax.fori_loop(0, ROWS_PER_WORKER // LANES, group_body, 0)
    pltpu.sync_copy(out_v, thr_hbm.at[pl.ds(base_row, ROWS_PER_WORKER)])


_sc_thresholds = functools.partial(
    pl.kernel,
    mesh=plsc.VectorSubcoreMesh(core_axis_name="c", subcore_axis_name="s"),
    out_type=jax.ShapeDtypeStruct((ROWS,), jnp.int32),
    compiler_params=pltpu.CompilerParams(needs_layout_passes=False),
    scratch_types=[
        pltpu.VMEM((COLS,), jnp.float32),
        pltpu.VMEM((COLS,), jnp.int32),
        pltpu.VMEM((COLS + LANES,), jnp.int32),
        pltpu.VMEM((GROUPS * BLK,), jnp.int32),
        pltpu.VMEM((ROWS_PER_WORKER,), jnp.int32),
    ],
)(_sc_body)


def _tc_body(z_ref, thr_ref, o_ref):
    z = z_ref[...]
    b = lax.bitcast_convert_type(z, jnp.int32)
    k = _keys_from_bits(b)
    kth = thr_ref[...]
    sig = jax.nn.sigmoid(z * jnp.float32(SIG_SCALE))
    o_ref[...] = jnp.where(k <= kth, jnp.float32(0.0), sig)


@jax.jit
def _impl(z):
    thr = _sc_thresholds(z)
    out = pl.pallas_call(
        _tc_body,
        grid=(ROWS // TC_BLOCK_ROWS,),
        in_specs=[
            pl.BlockSpec((TC_BLOCK_ROWS, COLS), lambda i: (i, 0)),
            pl.BlockSpec((TC_BLOCK_ROWS, 1), lambda i: (i, 0)),
        ],
        out_specs=pl.BlockSpec((TC_BLOCK_ROWS, COLS), lambda i: (i, 0)),
        out_shape=jax.ShapeDtypeStruct((ROWS, COLS), jnp.float32),
    )(z, thr.reshape(ROWS, 1))
    return out


def kernel(z_loga):
    return _impl(z_loga)


# speedup vs baseline: 3.6797x; 1.2332x over previous
import functools

import jax
import jax.numpy as jnp
from jax import lax
from jax.experimental import pallas as pl
from jax.experimental.pallas import tpu as pltpu
from jax.experimental.pallas import tpu_sc as plsc

ROWS, COLS = 2048, 4096
NUM_ZERO = 2048
LANES = 16
NCHUNK = COLS // LANES
NWORKERS = 32
ROWS_PER_WORKER = ROWS // NWORKERS
NBUCKET = 256
NGROUP = NBUCKET // LANES
GROUPS = 4
GCH = NCHUNK // GROUPS
BLK = GCH * LANES + LANES
UNROLL = 8
SIG_SCALE = 0.8 / (2.0 / 3.0)
TC_BLOCK_ROWS = 128
I32_MIN = -2147483648
I32_MAX = 2147483647
MAX_ROUNDS = 64


def _keys_from_bits(b):
    return b ^ ((b >> 31) & 0x7FFFFFFF)


def _sc_body(z_hbm, thr_hbm, row_v, key_v, cand_a, cand_b, hist_v, out_v):
    cid = lax.axis_index("c")
    sid = lax.axis_index("s")
    wid = sid * 2 + cid
    base_row = wid * ROWS_PER_WORKER
    ones = jnp.ones((LANES,), jnp.int32)
    zeros = jnp.zeros((LANES,), jnp.int32)
    lane_iota = lax.iota(jnp.int32, LANES)
    lane_base = lane_iota * NBUCKET

    def group_counts(g):
        acc = hist_v[pl.ds(g * LANES, LANES)]
        for l in range(1, LANES):
            acc = acc + hist_v[pl.ds(l * NBUCKET + g * LANES, LANES)]
        return acc

    def select_round(src, dst, n, t):
        sample = src[pl.ds(0, LANES)]
        ss, _ = plsc.sort_key_val(sample, sample)
        p1, p2, p3 = ss[4], ss[8], ss[12]
        nch = (n + LANES - 1) >> 4

        def cbody(j, accs):
            a1, a2, a3, e1, e2, e3 = accs
            k = src[pl.ds(j * LANES, LANES)]
            k = jnp.where(lane_iota < (n - j * LANES), k, I32_MAX)
            a1 = a1 + (k < p1).astype(jnp.int32)
            a2 = a2 + (k < p2).astype(jnp.int32)
            a3 = a3 + (k < p3).astype(jnp.int32)
            e1 = e1 + (k == p1).astype(jnp.int32)
            e2 = e2 + (k == p2).astype(jnp.int32)
            e3 = e3 + (k == p3).astype(jnp.int32)
            return a1, a2, a3, e1, e2, e3

        accs = lax.fori_loop(0, nch, cbody, (zeros,) * 6)
        clt1, clt2, clt3 = (jnp.sum(a) for a in accs[:3])
        ceq1, ceq2, ceq3 = (jnp.sum(a) for a in accs[3:])
        cle1 = clt1 + ceq1
        cle2 = clt2 + ceq2
        cle3 = clt3 + ceq3

        in1 = jnp.logical_and(t > clt1, t <= cle1)
        in2 = jnp.logical_and(t > clt2, t <= cle2)
        in3 = jnp.logical_and(t > clt3, t <= cle3)
        done = jnp.logical_or(in1, jnp.logical_or(in2, in3))
        ans = jnp.where(t <= cle1, p1, jnp.where(t <= cle2, p2, p3))
        lo = jnp.where(t <= clt1, I32_MIN,
                       jnp.where(t <= clt2, p1,
                                 jnp.where(t <= clt3, p2, p3)))
        hi = jnp.where(t <= clt1, p1,
                       jnp.where(t <= clt2, p2,
                                 jnp.where(t <= clt3, p3, I32_MAX)))
        below = jnp.where(t <= clt1, 0,
                          jnp.where(t <= clt2, cle1,
                                    jnp.where(t <= clt3, cle2, cle3)))

        def compact(j, ptr):
            k = src[pl.ds(j * LANES, LANES)]
            m = jnp.logical_and(lane_iota < (n - j * LANES),
                                jnp.logical_and(k > lo, k < hi))
            plsc.store_compressed(dst.at[pl.ds(ptr, LANES)], k, mask=m)
            return ptr + plsc.all_reduce_population_count(m)[0]

        ptr = lax.fori_loop(0, nch, compact, jnp.int32(0))
        n2 = jnp.where(done, 1, ptr)
        t2 = jnp.where(done, 1, t - below)
        return n2, t2, done.astype(jnp.int32), ans

    def row_threshold(row):
        pltpu.sync_copy(z_hbm.at[row], row_v)

        for c in range(NBUCKET * LANES // LANES):
            hist_v[pl.ds(c * LANES, LANES)] = zeros

        def pass1(jo, _):
            for ji in range(UNROLL):
                j = jo * UNROLL + ji
                z = row_v[pl.ds(j * LANES, LANES)]
                b = lax.bitcast_convert_type(z, jnp.int32)
                k = _keys_from_bits(b)
                key_v[pl.ds(j * LANES, LANES)] = k
                bk = ((k >> 24) + 128) + lane_base
                plsc.addupdate_scatter(hist_v, [bk], ones)
            return 0

        lax.fori_loop(0, NCHUNK // UNROLL, pass1, 0)

        kt = jnp.int32(NUM_ZERO)

        def scan_group(g, carry):
            found, selg, cumsel, cum = carry
            h = group_counts(g)
            cs = plsc.cumsum(h)
            tot = cs[15]
            need = kt - cum
            ff = plsc.all_reduce_population_count(cs < need)[0]
            hit = (1 - found) * jnp.where(ff < LANES, 1, 0)
            selg = jnp.where(hit == 1, g, selg)
            cumsel = jnp.where(hit == 1, cum, cumsel)
            found = jnp.where(ff < LANES, 1, found)
            cum = cum + tot
            return found, selg, cumsel, cum

        _, selg, cumsel, _ = lax.fori_loop(
            0, NGROUP, scan_group,
            (jnp.int32(0), jnp.int32(0), jnp.int32(0), jnp.int32(0)))

        h = group_counts(selg)
        cs = plsc.cumsum(h)
        need = kt - cumsel
        pref_mask = cs < need
        ff = plsc.all_reduce_population_count(pref_mask)[0]
        below_in = jnp.sum(jnp.where(pref_mask, h, 0))
        sel = selg * LANES + ff
        prefix = sel - 128
        t = kt - (cumsel + below_in)

        def compact1(jj, ptrs):
            new_ptrs = []
            for gi in range(GROUPS):
                ptr = ptrs[gi]
                j = gi * GCH + jj
                k = key_v[pl.ds(j * LANES, LANES)]
                pm = (k >> 24) == prefix
                plsc.store_compressed(
                    cand_b.at[pl.ds(gi * BLK + ptr, LANES)], k, mask=pm)
                new_ptrs.append(
                    ptr + plsc.all_reduce_population_count(pm)[0])
            return tuple(new_ptrs)

        ptrs = lax.fori_loop(0, GCH, compact1, (jnp.int32(0),) * GROUPS)

        q = jnp.int32(0)
        for gi in range(GROUPS):
            s_gi = ptrs[gi]

            def mcopy(c, _, gi=gi, q=q):
                cand_a[pl.ds(q + c * LANES, LANES)] = \
                    cand_b[pl.ds(gi * BLK + c * LANES, LANES)]
                return 0

            lax.fori_loop(0, (s_gi + LANES - 1) >> 4, mcopy, 0)
            q = q + s_gi
        n = q

        def wcond(c):
            n, t, done, ans, it = c
            return jnp.logical_and(
                done == 0, jnp.logical_and(n > LANES, it < MAX_ROUNDS))

        def wbody(c):
            n, t, done, ans, it = c
            n2, t2, d2, a2 = select_round(cand_a, cand_b, n, t)

            def copyb(j, _):
                cand_a[pl.ds(j * LANES, LANES)] = \
                    cand_b[pl.ds(j * LANES, LANES)]
                return 0

            lax.fori_loop(0, (n2 + LANES - 1) >> 4, copyb, 0)
            return n2, t2, d2, jnp.where(d2 == 1, a2, ans), it + 1

        n, t, done, ans, _ = lax.while_loop(
            wcond, wbody, (n, t, jnp.int32(0), jnp.int32(0), jnp.int32(0)))

        s = jnp.where(lane_iota < n, cand_a[pl.ds(0, LANES)], I32_MAX)
        ssort, _ = plsc.sort_key_val(s, s)
        pick = jnp.sum(jnp.where(lane_iota == (t - 1), ssort, 0))
        return jnp.where(done == 1, ans, pick)

    def group_body(g, _):
        def row_body(i, acc):
            kth = row_threshold(base_row + g * LANES + i)
            return jnp.where(lane_iota == i, kth, acc)

        acc = lax.fori_loop(0, LANES, row_body, zeros)
        out_v[pl.ds(g * LANES, LANES)] = acc
        return 0

    lax.fori_loop(0, ROWS_PER_WORKER // LANES, group_body, 0)
    pltpu.sync_copy(out_v, thr_hbm.at[pl.ds(base_row, ROWS_PER_WORKER)])


_sc_thresholds = functools.partial(
    pl.kernel,
    mesh=plsc.VectorSubcoreMesh(core_axis_name="c", subcore_axis_name="s"),
    out_type=jax.ShapeDtypeStruct((ROWS,), jnp.int32),
    compiler_params=pltpu.CompilerParams(needs_layout_passes=False),
    scratch_types=[
        pltpu.VMEM((COLS,), jnp.float32),
        pltpu.VMEM((COLS + LANES,), jnp.int32),
        pltpu.VMEM((COLS + LANES,), jnp.int32),
        pltpu.VMEM((GROUPS * BLK,), jnp.int32),
        pltpu.VMEM((NBUCKET * LANES,), jnp.int32),
        pltpu.VMEM((ROWS_PER_WORKER,), jnp.int32),
    ],
)(_sc_body)


def _tc_body(z_ref, thr_ref, o_ref):
    z = z_ref[...]
    b = lax.bitcast_convert_type(z, jnp.int32)
    k = _keys_from_bits(b)
    kth = thr_ref[...]
    sig = jax.nn.sigmoid(z * jnp.float32(SIG_SCALE))
    o_ref[...] = jnp.where(k <= kth, jnp.float32(0.0), sig)


@jax.jit
def _impl(z):
    thr = _sc_thresholds(z)
    out = pl.pallas_call(
        _tc_body,
        grid=(ROWS // TC_BLOCK_ROWS,),
        in_specs=[
            pl.BlockSpec((TC_BLOCK_ROWS, COLS), lambda i: (i, 0)),
            pl.BlockSpec((TC_BLOCK_ROWS, 1), lambda i: (i, 0)),
        ],
        out_specs=pl.BlockSpec((TC_BLOCK_ROWS, COLS), lambda i: (i, 0)),
        out_shape=jax.ShapeDtypeStruct((ROWS, COLS), jnp.float32),
    )(z, thr.reshape(ROWS, 1))
    return out


def kernel(z_loga):
    return _impl(z_loga)
